# baseline (speedup 1.0000x reference)
import functools

import jax
import jax.numpy as jnp
from jax import lax
from jax.experimental import pallas as pl
from jax.experimental.pallas import tpu as pltpu
from jax.experimental.pallas import tpu_sc as plsc

_NC = 2
_NS = 16
_NW = _NC * _NS
_L = 16

_BT = 128
_NBUF = 5


@jax.jit
def _sc_embed(idx_lin, W, pos_enc):
    ST, NBT, SR, BT = idx_lin.shape
    S = ST * SR
    D = W.shape[1]
    NSG = _NW // NBT
    spg = S // NSG
    n_st = spg // SR + 1
    DT = D // 8

    mesh = plsc.VectorSubcoreMesh(
        core_axis_name="c", subcore_axis_name="s",
        num_cores=_NC, num_subcores=_NS)

    @functools.partial(
        pl.kernel,
        out_type=jax.ShapeDtypeStruct((S, DT, NBT, 8, BT), jnp.float32),
        mesh=mesh,
        scratch_types=[
            pltpu.VMEM((n_st, SR, BT), jnp.int32),
            pltpu.VMEM((S, D), jnp.float32),
            pltpu.VMEM((_NBUF, BT, D), jnp.float32),
            pltpu.VMEM((_NBUF, DT, 8, BT + 1), jnp.float32),
        ] + [pltpu.SemaphoreType.DMA] * (2 * _NBUF),
        compiler_params=pltpu.CompilerParams(
            use_tc_tiling_on_sc=False, needs_layout_passes=False),
    )
    def body(w_hbm, idx_hbm, pos_hbm, out_hbm, idx_v, pos_v, gbuf, obuf,
             *sems):
        gsems = sems[:_NBUF]
        osems = sems[_NBUF:]
        wid = lax.axis_index("s") * _NC + lax.axis_index("c")
        bt = lax.rem(wid, NBT)
        sg = lax.div(wid, NBT)
        s_base = sg * spg
        st_lo = lax.div(s_base, SR)

        lane = lax.iota(jnp.int32, _L)
        di_idx = [lax.div(lane + j * _L, 8) for j in range(D // _L)]
        dr_idx = [lax.rem(lane + j * _L, 8) for j in range(D // _L)]
        zero = lane * 0

        def gather(k, b):
            s = s_base + k
            t = lax.div(s, SR) - st_lo
            sr = lax.rem(s, SR)
            pltpu.make_async_copy(
                w_hbm.at[idx_v.at[t, sr]], gbuf.at[b], gsems[b]).start()

        def gather_wait(b):
            pltpu.make_async_copy(
                w_hbm.at[idx_v.at[0, 0]], gbuf.at[b], gsems[b]).wait()

        def store(k, b):
            s = s_base + k
            pltpu.make_async_copy(
                obuf.at[b, :, :, pl.ds(0, BT)],
                out_hbm.at[s, :, bt], osems[b]).start()

        def store_wait(b):
            pltpu.make_async_copy(
                obuf.at[b, :, :, pl.ds(0, BT)],
                out_hbm.at[s_base, :, bt], osems[b]).wait()

        for t in range(n_st):
            pltpu.make_async_copy(
                idx_hbm.at[st_lo + t, bt], idx_v.at[t], gsems[0]).start()
        pltpu.make_async_copy(pos_hbm, pos_v, gsems[1]).start()
        for t in range(n_st):
            pltpu.make_async_copy(
                idx_hbm.at[st_lo, bt], idx_v.at[0], gsems[0]).wait()
        pltpu.make_async_copy(pos_hbm, pos_v, gsems[1]).wait()

        for b in range(_NBUF):
            gather(b, b)

        @pl.loop(0, spg // _NBUF)
        def _outer(k0):
            for b in range(_NBUF):
                k = k0 * _NBUF + b
                s = s_base + k
                gather_wait(b)

                @pl.when(k0 > 0)
                def _():
                    store_wait(b)

                pvecs = [pos_v[s, pl.ds(j * _L, _L)] for j in range(D // _L)]

                @pl.loop(0, BT, unroll=8)
                def _row(r):
                    rvec = zero + r
                    for j in range(D // _L):
                        v = gbuf[b, r, pl.ds(j * _L, _L)] + pvecs[j]
                        plsc.store_scatter(
                            obuf.at[b], [di_idx[j], dr_idx[j], rvec], v)

                store(k, b)

                @pl.when(k + _NBUF < spg)
                def _():
                    gather(k + _NBUF, b)

        for b in range(_NBUF):
            store_wait(b)

    return body(W, idx_lin, pos_enc)


def kernel(inputs, W, pos_enc):
    B, S = inputs.shape
    V, D = W.shape
    assert B % _BT == 0 and S % 8 == 0
    NBT = B // _BT
    NSG = _NW // NBT
    assert S % NSG == 0 and (S // NSG) % _NBUF == 0
    assert D % _L == 0 and D % 8 == 0

    idx_lin = inputs.reshape(NBT, _BT, S // 8, 8).transpose(2, 0, 3, 1)
    out = _sc_embed(idx_lin, W, pos_enc)
    out = out.transpose(0, 1, 3, 2, 4)
    out = out.reshape(S, D, B)
    return out.transpose(2, 0, 1)


# speedup vs baseline: 1.6851x; 1.6851x over previous
import functools

import jax
import jax.numpy as jnp
from jax import lax
from jax.experimental import pallas as pl
from jax.experimental.pallas import tpu as pltpu
from jax.experimental.pallas import tpu_sc as plsc

_NC = 2
_NS = 16
_NW = _NC * _NS
_L = 16

_BT = 128
_NBUF = 5


@jax.jit
def _sc_embed(idx_lin, W, pos_enc):
    ST, NBT, SR, BT = idx_lin.shape
    S = ST * SR
    D = W.shape[1]
    NSG = _NW // NBT
    spg = S // NSG
    n_st = spg // SR + 1
    DT = D // 8

    mesh = plsc.VectorSubcoreMesh(
        core_axis_name="c", subcore_axis_name="s",
        num_cores=_NC, num_subcores=_NS)

    @functools.partial(
        pl.kernel,
        out_type=jax.ShapeDtypeStruct((S, DT, NBT, 8, BT), jnp.float32),
        mesh=mesh,
        scratch_types=[
            pltpu.VMEM((n_st, SR, BT), jnp.int32),
            pltpu.VMEM((S, D), jnp.float32),
            pltpu.VMEM((_NBUF, BT, D), jnp.float32),
            pltpu.VMEM((_NBUF, DT, 8, BT + 1), jnp.float32),
        ] + [pltpu.SemaphoreType.DMA] * (2 * _NBUF),
        compiler_params=pltpu.CompilerParams(
            use_tc_tiling_on_sc=False, needs_layout_passes=False),
    )
    def body(w_hbm, idx_hbm, pos_hbm, out_hbm, idx_v, pos_v, gbuf, obuf,
             *sems):
        gsems = sems[:_NBUF]
        osems = sems[_NBUF:]
        wid = lax.axis_index("s") * _NC + lax.axis_index("c")
        bt = lax.rem(wid, NBT)
        sg = lax.div(wid, NBT)
        s_base = sg * spg
        st_lo = lax.div(s_base, SR)

        lane = lax.iota(jnp.int32, _L)
        di_idx = [lax.div(lane + j * _L, 8) for j in range(D // _L)]
        dr_idx = [lax.rem(lane + j * _L, 8) for j in range(D // _L)]
        zero = lane * 0

        def gather(k, b):
            s = s_base + k
            t = lax.div(s, SR) - st_lo
            sr = lax.rem(s, SR)
            pltpu.make_async_copy(
                w_hbm.at[idx_v.at[t, sr]], gbuf.at[b], gsems[b]).start()

        def gather_wait(b):
            pltpu.make_async_copy(
                w_hbm.at[idx_v.at[0, 0]], gbuf.at[b], gsems[b]).wait()

        def store(k, b):
            s = s_base + k
            pltpu.make_async_copy(
                obuf.at[b, :, :, pl.ds(0, BT)],
                out_hbm.at[s, :, bt], osems[b]).start()

        def store_wait(b):
            pltpu.make_async_copy(
                obuf.at[b, :, :, pl.ds(0, BT)],
                out_hbm.at[s_base, :, bt], osems[b]).wait()

        for t in range(n_st):
            pltpu.make_async_copy(
                idx_hbm.at[st_lo + t, bt], idx_v.at[t], gsems[0]).start()
        pltpu.make_async_copy(pos_hbm, pos_v, gsems[1]).start()
        for t in range(n_st):
            pltpu.make_async_copy(
                idx_hbm.at[st_lo, bt], idx_v.at[0], gsems[0]).wait()
        pltpu.make_async_copy(pos_hbm, pos_v, gsems[1]).wait()

        for b in range(_NBUF):
            gather(b, b)

        @pl.loop(0, spg // _NBUF)
        def _outer(k0):
            for b in range(_NBUF):
                k = k0 * _NBUF + b
                s = s_base + k
                gather_wait(b)

                @pl.when(k0 > 0)
                def _():
                    store_wait(b)

                pvecs = [pos_v[s, pl.ds(j * _L, _L)] for j in range(D // _L)]

                @plsc.parallel_loop(0, BT, unroll=8)
                def _row(r):
                    rvec = zero + r
                    for j in range(D // _L):
                        v = gbuf[b, r, pl.ds(j * _L, _L)] + pvecs[j]
                        plsc.store_scatter(
                            obuf.at[b], [di_idx[j], dr_idx[j], rvec], v)

                store(k, b)

                @pl.when(k + _NBUF < spg)
                def _():
                    gather(k + _NBUF, b)

        for b in range(_NBUF):
            store_wait(b)

    return body(W, idx_lin, pos_enc)


def kernel(inputs, W, pos_enc):
    B, S = inputs.shape
    V, D = W.shape
    assert B % _BT == 0 and S % 8 == 0
    NBT = B // _BT
    NSG = _NW // NBT
    assert S % NSG == 0 and (S // NSG) % _NBUF == 0
    assert D % _L == 0 and D % 8 == 0

    idx_lin = inputs.reshape(NBT, _BT, S // 8, 8).transpose(2, 0, 3, 1)
    out = _sc_embed(idx_lin, W, pos_enc)
    out = out.transpose(0, 1, 3, 2, 4)
    out = out.reshape(S, D, B)
    return out.transpose(2, 0, 1)
